# baseline (speedup 1.0000x reference)
import functools

import jax
import jax.numpy as jnp
from jax import lax
from jax.experimental import pallas as pl
from jax.experimental.pallas import tpu as pltpu
from jax.experimental.pallas import tpu_sc as plsc

N_NODES = 10000
N_EDGES = 320000
H = 128
HW = H // 2
L = 16
NW = 32
EPW = N_EDGES // NW
CHH = 80
CHS = 64
NH = 21
NS = 130
NT = NH + NS
EH = NH * CHH
RS = 128
NSLOT = 3


def _tables_body(h_ref, wa_ref, wb_ref, b1_ref, a_ref, b_ref):
    x = h_ref[...]
    a_ref[...] = (
        jnp.dot(x, wa_ref[...], preferred_element_type=jnp.float32) + b1_ref[...]
    )
    b_ref[...] = jnp.dot(x, wb_ref[...], preferred_element_type=jnp.float32)


def _node_tables(h, W1, b1):
    rows = 1000
    grid = (N_NODES // rows,)
    return pl.pallas_call(
        _tables_body,
        grid=grid,
        in_specs=[
            pl.BlockSpec((rows, H), lambda i: (i, 0)),
            pl.BlockSpec((H, H), lambda i: (0, 0)),
            pl.BlockSpec((H, H), lambda i: (0, 0)),
            pl.BlockSpec((1, H), lambda i: (0, 0)),
        ],
        out_specs=[
            pl.BlockSpec((rows, H), lambda i: (i, 0)),
            pl.BlockSpec((rows, H), lambda i: (i, 0)),
        ],
        out_shape=[
            jax.ShapeDtypeStruct((N_NODES, H), jnp.float32),
            jax.ShapeDtypeStruct((N_NODES, H), jnp.float32),
        ],
    )(h, W1[:H], W1[H:], b1.reshape(1, H))


def _pack_combined(a_tab, b_tab):
    ua = lax.bitcast_convert_type(a_tab.astype(jnp.bfloat16), jnp.uint16).astype(jnp.uint32)
    ub = lax.bitcast_convert_type(b_tab.astype(jnp.bfloat16), jnp.uint16).astype(jnp.uint32)
    pa = (ua[:, :HW] << 16) | ua[:, HW:]
    pb = (ub[:, :HW] << 16) | ub[:, HW:]
    return lax.bitcast_convert_type(jnp.concatenate([pa, pb], axis=1), jnp.float32)


def _permute(a, perm):
    return lax.gather(
        a, perm[:, None],
        lax.GatherDimensionNumbers(
            offset_dims=(), collapsed_slice_dims=(0,), start_index_map=(0,)
        ),
        slice_sizes=(1,),
        mode=lax.GatherScatterMode.PROMISE_IN_BOUNDS,
        unique_indices=True, indices_are_sorted=False,
    )


def _bc_i(x):
    return lax.bitcast_convert_type(x, jnp.int32)


def _bc_f(x):
    return lax.bitcast_convert_type(x, jnp.float32)


def _h_before(t):
    return (t * NH) // NT


def _is_h(t):
    return ((t + 1) * NH) // NT - (t * NH) // NT == 1


@functools.partial(
    pl.kernel,
    out_type=jax.ShapeDtypeStruct((N_EDGES,), jnp.float32),
    mesh=plsc.VectorSubcoreMesh(core_axis_name="c", subcore_axis_name="s"),
    scratch_types=[
        pltpu.VMEM_SHARED((N_NODES, H), jnp.float32),
        pltpu.VMEM((NSLOT, RS, H), jnp.float32),
        pltpu.VMEM((NSLOT, CHH), jnp.int32),
        pltpu.VMEM((NSLOT, CHH), jnp.int32),
        pltpu.VMEM((H,), jnp.float32),
        pltpu.VMEM((2, CHH), jnp.float32),
        pltpu.SemaphoreType.DMA,
        pltpu.SemaphoreType.DMA,
        pltpu.SemaphoreType.DMA,
        pltpu.SemaphoreType.DMA,
    ],
)
def _edge_decode(a_hbm, b_hbm, cpk_hbm, src_hbm, dst_hbm, w2_hbm, out_hbm,
                 spt, ring, src_i, dst_i, w2_v, out_b, semP0, semP1, semI, semO):
    sid = lax.axis_index("s")
    wid = sid * 2 + lax.axis_index("c")
    base0 = wid * EPW

    rows_per = 1000

    @pl.when(sid < N_NODES // rows_per)
    def _():
        pltpu.sync_copy(cpk_hbm.at[pl.ds(sid * rows_per, rows_per)],
                        spt.at[pl.ds(sid * rows_per, rows_per)])

    plsc.subcore_barrier()

    pltpu.sync_copy(w2_hbm, w2_v)
    w2r = [w2_v[pl.ds(k * L, L)] for k in range(H // L)]
    lane_ids = lax.iota(jnp.int32, L)
    perms = [(lane_ids + sh) & 15 for sh in (8, 4, 2, 1)]
    zero = jnp.zeros((L,), jnp.float32)

    def hsum_to_lane(acc, red, i):
        for p in perms:
            acc = acc + _permute(acc, p)
        return jnp.where(lane_ids == i, acc, red)

    def _dual(t, fn):
        if isinstance(t, int):
            fn(semP0 if t % 2 == 0 else semP1)
            return

        @pl.when(t % 2 == 0)
        def _():
            fn(semP0)

        @pl.when(t % 2 == 1)
        def _():
            fn(semP1)

    def _branch(is_h, fn_h, fn_s):
        if isinstance(is_h, bool):
            (fn_h if is_h else fn_s)()
            return

        pl.when(is_h)(fn_h)
        pl.when(jnp.logical_not(is_h))(fn_s)

    def issue_idx(t, slot):
        base_h = base0 + _h_before(t) * CHH
        base_s = base0 + EH + (t - _h_before(t)) * CHS

        def go_h():
            pltpu.async_copy(src_hbm.at[pl.ds(base_h, CHH)], src_i.at[slot], semI)
            pltpu.async_copy(dst_hbm.at[pl.ds(base_h, CHH)], dst_i.at[slot], semI)

        def go_s():
            pltpu.async_copy(src_hbm.at[pl.ds(base_s, CHS)],
                             src_i.at[slot, pl.ds(0, CHS)], semI)
            pltpu.async_copy(dst_hbm.at[pl.ds(base_s, CHS)],
                             dst_i.at[slot, pl.ds(0, CHS)], semI)

        _branch(_is_h(t), go_h, go_s)

    def wait_idx(t, slot):
        def go_h():
            pltpu.make_async_copy(src_hbm.at[pl.ds(0, CHH)], src_i.at[slot], semI).wait()
            pltpu.make_async_copy(dst_hbm.at[pl.ds(0, CHH)], dst_i.at[slot], semI).wait()

        def go_s():
            pltpu.make_async_copy(src_hbm.at[pl.ds(0, CHS)],
                                  src_i.at[slot, pl.ds(0, CHS)], semI).wait()
            pltpu.make_async_copy(dst_hbm.at[pl.ds(0, CHS)],
                                  dst_i.at[slot, pl.ds(0, CHS)], semI).wait()

        _branch(_is_h(t), go_h, go_s)

    def issue_gathers(t, slot, is_h):
        def go_h():
            _dual(t, lambda s: pltpu.async_copy(
                a_hbm.at[src_i.at[slot]], ring.at[slot, pl.ds(0, CHH)], s))

        def go_s():
            def go(s):
                pltpu.async_copy(spt.at[src_i.at[slot, pl.ds(0, CHS)]],
                                 ring.at[slot, pl.ds(0, CHS)], s)
                pltpu.async_copy(spt.at[dst_i.at[slot, pl.ds(0, CHS)]],
                                 ring.at[slot, pl.ds(CHS, CHS)], s)
            _dual(t, go)

        _branch(is_h, go_h, go_s)

    def issue_gb(t, slot):
        _dual(t, lambda s: pltpu.async_copy(
            b_hbm.at[dst_i.at[slot]], ring.at[slot, pl.ds(0, CHH)], s, add=True))

    def drain_h(t, slot):
        _dual(t, lambda s: pltpu.make_async_copy(
            a_hbm.at[pl.ds(0, CHH)], ring.at[slot, pl.ds(0, CHH)], s).wait())

    def drain_s(t, slot):
        def go(s):
            pltpu.make_async_copy(spt.at[pl.ds(0, CHS)],
                                  ring.at[slot, pl.ds(0, CHS)], s).wait()
            pltpu.make_async_copy(spt.at[pl.ds(0, CHS)],
                                  ring.at[slot, pl.ds(CHS, CHS)], s).wait()
        _dual(t, go)

    def compute_h(t, slot, oslot):
        def group_body(g, gc):
            e0 = g * L
            red = zero
            for i in range(L):
                acc = zero
                for k in range(H // L):
                    z = ring[slot, e0 + i, pl.ds(k * L, L)]
                    acc = acc + jnp.maximum(z, 0.0) * w2r[k]
                red = hsum_to_lane(acc, red, i)
            out_b[oslot, pl.ds(e0, L)] = red
            return gc

        lax.fori_loop(0, CHH // L, group_body, 0)
        base = base0 + _h_before(t) * CHH
        pltpu.async_copy(out_b.at[oslot], out_hbm.at[pl.ds(base, CHH)], semO)

    def compute_s(t, slot, oslot):
        def group_body(g, gc):
            e0 = g * L
            red = zero
            for i in range(L):
                acc_h = zero
                acc_l = zero
                for k in range(HW // L):
                    aw = ring[slot, e0 + i, pl.ds(k * L, L)]
                    bw = ring[slot, CHS + e0 + i, pl.ds(HW + k * L, L)]
                    zh = aw + bw
                    zl = _bc_f(_bc_i(aw) << 16) + _bc_f(_bc_i(bw) << 16)
                    acc_h = acc_h + jnp.maximum(zh, 0.0) * w2r[k]
                    acc_l = acc_l + jnp.maximum(zl, 0.0) * w2r[HW // L + k]
                red = hsum_to_lane(acc_h + acc_l, red, i)
            out_b[oslot, pl.ds(e0, L)] = red
            return gc

        lax.fori_loop(0, CHS // L, group_body, 0)
        base = base0 + EH + (t - _h_before(t)) * CHS
        pltpu.async_copy(out_b.at[oslot, pl.ds(0, CHS)],
                         out_hbm.at[pl.ds(base, CHS)], semO)

    def drain_out(is_h_tag):
        def go_h():
            pltpu.make_async_copy(out_b.at[0], out_hbm.at[pl.ds(0, CHH)], semO).wait()

        def go_s():
            pltpu.make_async_copy(out_b.at[0, pl.ds(0, CHS)],
                                  out_hbm.at[pl.ds(0, CHS)], semO).wait()

        _branch(is_h_tag, go_h, go_s)

    issue_idx(0, 0)
    issue_idx(1, 1)
    wait_idx(0, 0)
    issue_gathers(0, 0, _is_h(0))
    if _is_h(0):
        drain_h(0, 0)
        issue_gb(0, 0)
    wait_idx(1, 1)
    issue_gathers(1, 1, _is_h(1))
    issue_idx(2, 2)

    def body(t, carry):
        slot = t % NSLOT
        oslot = t % 2
        is_h_t = _is_h(t)

        @pl.when(jnp.logical_and(t < NT - 1, _is_h(t + 1)))
        def _():
            s1 = (t + 1) % NSLOT
            drain_h(t + 1, s1)
            issue_gb(t + 1, s1)

        @pl.when(is_h_t)
        def _():
            drain_h(t, slot)

        @pl.when(jnp.logical_not(is_h_t))
        def _():
            drain_s(t, slot)

        @pl.when(t < NT - 2)
        def _():
            s2 = (t + 2) % NSLOT
            wait_idx(t + 2, s2)
            issue_gathers(t + 2, s2, _is_h(t + 2))

        @pl.when(t < NT - 3)
        def _():
            issue_idx(t + 3, (t + 3) % NSLOT)

        @pl.when(t >= 2)
        def _():
            drain_out(_is_h(t - 2))

        @pl.when(is_h_t)
        def _():
            compute_h(t, slot, oslot)

        @pl.when(jnp.logical_not(is_h_t))
        def _():
            compute_s(t, slot, oslot)

        return carry

    lax.fori_loop(0, NT, body, 0)

    drain_out(_is_h(NT - 2))
    drain_out(_is_h(NT - 1))


def kernel(edges, h, W1, b1, W2, b2):
    edges = edges.astype(jnp.int32)
    a_tab, b_tab = _node_tables(h, W1, b1)
    out = _edge_decode(
        a_tab, b_tab, _pack_combined(a_tab, b_tab),
        edges[0], edges[1], W2.reshape(H),
    )
    return out + b2[0]


# speedup vs baseline: 1.0975x; 1.0975x over previous
import functools

import jax
import jax.numpy as jnp
from jax import lax
from jax.experimental import pallas as pl
from jax.experimental.pallas import tpu as pltpu
from jax.experimental.pallas import tpu_sc as plsc

N_NODES = 10000
N_EDGES = 320000
H = 128
HW = H // 2
L = 16
NW = 32
EPW = N_EDGES // NW
CHH = 80
CHS = 64
NH = 1
NS = 155
NT = NH + NS
EH = NH * CHH
RS = 128
NSLOT = 3


def _tables_body(h_ref, wa_ref, wb_ref, b1_ref, a_ref, b_ref):
    x = h_ref[...]
    a_ref[...] = (
        jnp.dot(x, wa_ref[...], preferred_element_type=jnp.float32) + b1_ref[...]
    )
    b_ref[...] = jnp.dot(x, wb_ref[...], preferred_element_type=jnp.float32)


def _node_tables(h, W1, b1):
    rows = 1000
    grid = (N_NODES // rows,)
    return pl.pallas_call(
        _tables_body,
        grid=grid,
        in_specs=[
            pl.BlockSpec((rows, H), lambda i: (i, 0)),
            pl.BlockSpec((H, H), lambda i: (0, 0)),
            pl.BlockSpec((H, H), lambda i: (0, 0)),
            pl.BlockSpec((1, H), lambda i: (0, 0)),
        ],
        out_specs=[
            pl.BlockSpec((rows, H), lambda i: (i, 0)),
            pl.BlockSpec((rows, H), lambda i: (i, 0)),
        ],
        out_shape=[
            jax.ShapeDtypeStruct((N_NODES, H), jnp.float32),
            jax.ShapeDtypeStruct((N_NODES, H), jnp.float32),
        ],
    )(h, W1[:H], W1[H:], b1.reshape(1, H))


def _pack_combined(a_tab, b_tab):
    ua = lax.bitcast_convert_type(a_tab.astype(jnp.bfloat16), jnp.uint16).astype(jnp.uint32)
    ub = lax.bitcast_convert_type(b_tab.astype(jnp.bfloat16), jnp.uint16).astype(jnp.uint32)
    pa = (ua[:, :HW] << 16) | ua[:, HW:]
    pb = (ub[:, :HW] << 16) | ub[:, HW:]
    return lax.bitcast_convert_type(jnp.concatenate([pa, pb], axis=1), jnp.float32)


def _permute(a, perm):
    return lax.gather(
        a, perm[:, None],
        lax.GatherDimensionNumbers(
            offset_dims=(), collapsed_slice_dims=(0,), start_index_map=(0,)
        ),
        slice_sizes=(1,),
        mode=lax.GatherScatterMode.PROMISE_IN_BOUNDS,
        unique_indices=True, indices_are_sorted=False,
    )


def _bc_i(x):
    return lax.bitcast_convert_type(x, jnp.int32)


def _bc_f(x):
    return lax.bitcast_convert_type(x, jnp.float32)


def _h_before(t):
    return (t * NH) // NT


def _is_h(t):
    return ((t + 1) * NH) // NT - (t * NH) // NT == 1


@functools.partial(
    pl.kernel,
    out_type=jax.ShapeDtypeStruct((N_EDGES,), jnp.float32),
    mesh=plsc.VectorSubcoreMesh(core_axis_name="c", subcore_axis_name="s"),
    scratch_types=[
        pltpu.VMEM_SHARED((N_NODES, H), jnp.float32),
        pltpu.VMEM((NSLOT, RS, H), jnp.float32),
        pltpu.VMEM((NSLOT, CHH), jnp.int32),
        pltpu.VMEM((NSLOT, CHH), jnp.int32),
        pltpu.VMEM((H,), jnp.float32),
        pltpu.VMEM((2, CHH), jnp.float32),
        pltpu.SemaphoreType.DMA,
        pltpu.SemaphoreType.DMA,
        pltpu.SemaphoreType.DMA,
        pltpu.SemaphoreType.DMA,
    ],
)
def _edge_decode(a_hbm, b_hbm, cpk_hbm, src_hbm, dst_hbm, w2_hbm, out_hbm,
                 spt, ring, src_i, dst_i, w2_v, out_b, semP0, semP1, semI, semO):
    sid = lax.axis_index("s")
    wid = sid * 2 + lax.axis_index("c")
    base0 = wid * EPW

    rows_per = 1000

    @pl.when(sid < N_NODES // rows_per)
    def _():
        pltpu.sync_copy(cpk_hbm.at[pl.ds(sid * rows_per, rows_per)],
                        spt.at[pl.ds(sid * rows_per, rows_per)])

    plsc.subcore_barrier()

    pltpu.sync_copy(w2_hbm, w2_v)
    w2r = [w2_v[pl.ds(k * L, L)] for k in range(H // L)]
    lane_ids = lax.iota(jnp.int32, L)
    perms = [(lane_ids + sh) & 15 for sh in (8, 4, 2, 1)]
    zero = jnp.zeros((L,), jnp.float32)

    def hsum_to_lane(acc, red, i):
        for p in perms:
            acc = acc + _permute(acc, p)
        return jnp.where(lane_ids == i, acc, red)

    def _dual(t, fn):
        if isinstance(t, int):
            fn(semP0 if t % 2 == 0 else semP1)
            return

        @pl.when(t % 2 == 0)
        def _():
            fn(semP0)

        @pl.when(t % 2 == 1)
        def _():
            fn(semP1)

    def _branch(is_h, fn_h, fn_s):
        if isinstance(is_h, bool):
            (fn_h if is_h else fn_s)()
            return

        pl.when(is_h)(fn_h)
        pl.when(jnp.logical_not(is_h))(fn_s)

    def issue_idx(t, slot):
        base_h = base0 + _h_before(t) * CHH
        base_s = base0 + EH + (t - _h_before(t)) * CHS

        def go_h():
            pltpu.async_copy(src_hbm.at[pl.ds(base_h, CHH)], src_i.at[slot], semI)
            pltpu.async_copy(dst_hbm.at[pl.ds(base_h, CHH)], dst_i.at[slot], semI)

        def go_s():
            pltpu.async_copy(src_hbm.at[pl.ds(base_s, CHS)],
                             src_i.at[slot, pl.ds(0, CHS)], semI)
            pltpu.async_copy(dst_hbm.at[pl.ds(base_s, CHS)],
                             dst_i.at[slot, pl.ds(0, CHS)], semI)

        _branch(_is_h(t), go_h, go_s)

    def wait_idx(t, slot):
        def go_h():
            pltpu.make_async_copy(src_hbm.at[pl.ds(0, CHH)], src_i.at[slot], semI).wait()
            pltpu.make_async_copy(dst_hbm.at[pl.ds(0, CHH)], dst_i.at[slot], semI).wait()

        def go_s():
            pltpu.make_async_copy(src_hbm.at[pl.ds(0, CHS)],
                                  src_i.at[slot, pl.ds(0, CHS)], semI).wait()
            pltpu.make_async_copy(dst_hbm.at[pl.ds(0, CHS)],
                                  dst_i.at[slot, pl.ds(0, CHS)], semI).wait()

        _branch(_is_h(t), go_h, go_s)

    def issue_gathers(t, slot, is_h):
        def go_h():
            _dual(t, lambda s: pltpu.async_copy(
                a_hbm.at[src_i.at[slot]], ring.at[slot, pl.ds(0, CHH)], s))

        def go_s():
            def go(s):
                pltpu.async_copy(spt.at[src_i.at[slot, pl.ds(0, CHS)]],
                                 ring.at[slot, pl.ds(0, CHS)], s)
                pltpu.async_copy(spt.at[dst_i.at[slot, pl.ds(0, CHS)]],
                                 ring.at[slot, pl.ds(CHS, CHS)], s)
            _dual(t, go)

        _branch(is_h, go_h, go_s)

    def issue_gb(t, slot):
        _dual(t, lambda s: pltpu.async_copy(
            b_hbm.at[dst_i.at[slot]], ring.at[slot, pl.ds(0, CHH)], s, add=True))

    def drain_h(t, slot):
        _dual(t, lambda s: pltpu.make_async_copy(
            a_hbm.at[pl.ds(0, CHH)], ring.at[slot, pl.ds(0, CHH)], s).wait())

    def drain_s(t, slot):
        def go(s):
            pltpu.make_async_copy(spt.at[pl.ds(0, CHS)],
                                  ring.at[slot, pl.ds(0, CHS)], s).wait()
            pltpu.make_async_copy(spt.at[pl.ds(0, CHS)],
                                  ring.at[slot, pl.ds(CHS, CHS)], s).wait()
        _dual(t, go)

    def compute_h(t, slot, oslot):
        def group_body(g, gc):
            e0 = g * L
            red = zero
            for i in range(L):
                acc = zero
                for k in range(H // L):
                    z = ring[slot, e0 + i, pl.ds(k * L, L)]
                    acc = acc + jnp.maximum(z, 0.0) * w2r[k]
                red = hsum_to_lane(acc, red, i)
            out_b[oslot, pl.ds(e0, L)] = red
            return gc

        lax.fori_loop(0, CHH // L, group_body, 0)
        base = base0 + _h_before(t) * CHH
        pltpu.async_copy(out_b.at[oslot], out_hbm.at[pl.ds(base, CHH)], semO)

    def compute_s(t, slot, oslot):
        def group_body(g, gc):
            e0 = g * L
            red = zero
            for i in range(L):
                acc_h = zero
                acc_l = zero
                for k in range(HW // L):
                    aw = ring[slot, e0 + i, pl.ds(k * L, L)]
                    bw = ring[slot, CHS + e0 + i, pl.ds(HW + k * L, L)]
                    zh = aw + bw
                    zl = _bc_f(_bc_i(aw) << 16) + _bc_f(_bc_i(bw) << 16)
                    acc_h = acc_h + jnp.maximum(zh, 0.0) * w2r[k]
                    acc_l = acc_l + jnp.maximum(zl, 0.0) * w2r[HW // L + k]
                red = hsum_to_lane(acc_h + acc_l, red, i)
            out_b[oslot, pl.ds(e0, L)] = red
            return gc

        lax.fori_loop(0, CHS // L, group_body, 0)
        base = base0 + EH + (t - _h_before(t)) * CHS
        pltpu.async_copy(out_b.at[oslot, pl.ds(0, CHS)],
                         out_hbm.at[pl.ds(base, CHS)], semO)

    def drain_out(is_h_tag):
        def go_h():
            pltpu.make_async_copy(out_b.at[0], out_hbm.at[pl.ds(0, CHH)], semO).wait()

        def go_s():
            pltpu.make_async_copy(out_b.at[0, pl.ds(0, CHS)],
                                  out_hbm.at[pl.ds(0, CHS)], semO).wait()

        _branch(is_h_tag, go_h, go_s)

    issue_idx(0, 0)
    issue_idx(1, 1)
    wait_idx(0, 0)
    issue_gathers(0, 0, _is_h(0))
    if _is_h(0):
        drain_h(0, 0)
        issue_gb(0, 0)
    wait_idx(1, 1)
    issue_gathers(1, 1, _is_h(1))
    issue_idx(2, 2)

    def body(t, carry):
        slot = t % NSLOT
        oslot = t % 2
        is_h_t = _is_h(t)

        @pl.when(jnp.logical_and(t < NT - 1, _is_h(t + 1)))
        def _():
            s1 = (t + 1) % NSLOT
            drain_h(t + 1, s1)
            issue_gb(t + 1, s1)

        @pl.when(is_h_t)
        def _():
            drain_h(t, slot)

        @pl.when(jnp.logical_not(is_h_t))
        def _():
            drain_s(t, slot)

        @pl.when(t < NT - 2)
        def _():
            s2 = (t + 2) % NSLOT
            wait_idx(t + 2, s2)
            issue_gathers(t + 2, s2, _is_h(t + 2))

        @pl.when(t < NT - 3)
        def _():
            issue_idx(t + 3, (t + 3) % NSLOT)

        @pl.when(t >= 2)
        def _():
            drain_out(_is_h(t - 2))

        @pl.when(is_h_t)
        def _():
            compute_h(t, slot, oslot)

        @pl.when(jnp.logical_not(is_h_t))
        def _():
            compute_s(t, slot, oslot)

        return carry

    lax.fori_loop(0, NT, body, 0)

    drain_out(_is_h(NT - 2))
    drain_out(_is_h(NT - 1))


def kernel(edges, h, W1, b1, W2, b2):
    edges = edges.astype(jnp.int32)
    a_tab, b_tab = _node_tables(h, W1, b1)
    out = _edge_decode(
        a_tab, b_tab, _pack_combined(a_tab, b_tab),
        edges[0], edges[1], W2.reshape(H),
    )
    return out + b2[0]
